# baseline (speedup 1.0000x reference)
import functools
import numpy as np
import jax
import jax.numpy as jnp
from jax import lax
from jax.experimental import pallas as pl
from jax.experimental.pallas import tpu as pltpu
from jax.experimental.pallas import tpu_sc as plsc

_C, _T, _H, _W = 3, 32, 256, 256
_NSLOW = 8
_IDX = np.linspace(0.0, _T - 1, _NSLOW).astype(np.int32)
_HH = _H // 2
_NTASK = _C * _T * 2
_NW = 32
_TPW = _NTASK // _NW

_mesh = plsc.VectorSubcoreMesh(core_axis_name="c", subcore_axis_name="s")


@functools.partial(
    pl.kernel,
    out_type=[
        jax.ShapeDtypeStruct((_C, _NSLOW, _H, _W), jnp.float32),
        jax.ShapeDtypeStruct((_C, _T, _H, _W), jnp.float32),
    ],
    mesh=_mesh,
    scratch_types=[
        pltpu.VMEM((_HH, _W), jnp.float32),
        pltpu.VMEM((_HH, _W), jnp.float32),
        pltpu.SemaphoreType.DMA,
        pltpu.SemaphoreType.DMA,
    ],
)
def _sc_pack(frames_hbm, slow_hbm, fast_hbm, buf0, buf1, sem_in, sem_out):
    wid = lax.axis_index("s") * 2 + lax.axis_index("c")
    bufs = (buf0, buf1)
    for k in range(_TPW):
        task = wid * _TPW + k
        c = task // (_T * 2)
        rem = task % (_T * 2)
        t = rem // 2
        half = rem % 2
        r0 = half * _HH
        buf = bufs[k % 2]
        pltpu.async_copy(frames_hbm.at[c, t, pl.ds(r0, _HH)], buf, sem_in).wait()
        pltpu.async_copy(buf, fast_hbm.at[c, t, pl.ds(r0, _HH)], sem_out).wait()
        s = (7 * t + 6) // 31
        sel = (31 * s) // 7 == t

        @pl.when(sel)
        def _():
            pltpu.async_copy(buf, slow_hbm.at[c, s, pl.ds(r0, _HH)], sem_out).wait()


def kernel(frames):
    slow, fast = _sc_pack(frames)
    return (slow, fast)


# speedup vs baseline: 1.2366x; 1.2366x over previous
import functools
import numpy as np
import jax
import jax.numpy as jnp
from jax import lax
from jax.experimental import pallas as pl
from jax.experimental.pallas import tpu as pltpu
from jax.experimental.pallas import tpu_sc as plsc

_C, _T, _H, _W = 3, 32, 256, 256
_NSLOW = 8
_IDX = np.linspace(0.0, _T - 1, _NSLOW).astype(np.int32)

_TB = 16


def _tc_body(in_ref, fast_ref):
    fast_ref[...] = in_ref[...]


def _tc_fast(frames):
    return pl.pallas_call(
        _tc_body,
        grid=(_C, _T // _TB),
        in_specs=[pl.BlockSpec((1, _TB, _H, _W), lambda c, q: (c, q, 0, 0))],
        out_specs=pl.BlockSpec((1, _TB, _H, _W), lambda c, q: (c, q, 0, 0)),
        out_shape=jax.ShapeDtypeStruct((_C, _T, _H, _W), jnp.float32),
    )(frames)


_QH = _H // 4
_NTASK = _C * _NSLOW * 4
_NW = 32
_TPW = _NTASK // _NW

_mesh = plsc.VectorSubcoreMesh(core_axis_name="c", subcore_axis_name="s")


@functools.partial(
    pl.kernel,
    out_type=jax.ShapeDtypeStruct((_C, _NSLOW, _H, _W), jnp.float32),
    mesh=_mesh,
    scratch_types=(
        [pltpu.VMEM((_QH, _W), jnp.float32) for _ in range(_TPW)]
        + [pltpu.SemaphoreType.DMA for _ in range(_TPW)]
        + [pltpu.SemaphoreType.DMA]
    ),
)
def _sc_slow(frames_hbm, slow_hbm, b0, b1, b2, s0, s1, s2, sem_out):
    wid = lax.axis_index("s") * 2 + lax.axis_index("c")
    bufs, sems = (b0, b1, b2), (s0, s1, s2)
    reads, tcs = [], []
    for k in range(_TPW):
        task = wid * _TPW + k
        f = task // 4
        qtr = task % 4
        c = f // _NSLOW
        s = f % _NSLOW
        t = (31 * s) // 7
        r0 = qtr * _QH
        cp = pltpu.make_async_copy(
            frames_hbm.at[c, t, pl.ds(r0, _QH)], bufs[k], sems[k])
        cp.start()
        reads.append(cp)
        tcs.append((c, s, r0))
    writes = []
    for k in range(_TPW):
        reads[k].wait()
        c, s, r0 = tcs[k]
        cp = pltpu.make_async_copy(
            bufs[k], slow_hbm.at[c, s, pl.ds(r0, _QH)], sem_out)
        cp.start()
        writes.append(cp)
    for cp in writes:
        cp.wait()


def kernel(frames):
    slow = _sc_slow(frames)
    fast = _tc_fast(frames)
    return (slow, fast)


# speedup vs baseline: 2.6591x; 2.1502x over previous
import numpy as np
import jax
import jax.numpy as jnp
from jax.experimental import pallas as pl

_C, _T, _H, _W = 3, 32, 256, 256
_ALPHA = 4
_NSLOW = _T // _ALPHA
_IDX = np.linspace(0.0, _T - 1, _NSLOW).astype(np.int32)
_TB = 32
_NQ = _T // _TB
_SPB = _NSLOW // _NQ
for _q in range(_NQ):
    for _j in range(_SPB):
        assert _TB * _q <= _IDX[_SPB * _q + _j] < _TB * (_q + 1)


def _body(in_ref, slow_ref, fast_ref):
    q = pl.program_id(1)
    fast_ref[...] = in_ref[...]
    for j in range(_SPB):
        i = _SPB * q + j
        off = (31 * i) // 7 - _TB * q
        slow_ref[:, pl.ds(j, 1)] = in_ref[:, pl.ds(off, 1)]


def kernel(frames):
    slow, fast = pl.pallas_call(
        _body,
        grid=(_C, _NQ),
        in_specs=[pl.BlockSpec((1, _TB, _H, _W), lambda c, q: (c, q, 0, 0))],
        out_specs=[
            pl.BlockSpec((1, _SPB, _H, _W), lambda c, q: (c, q, 0, 0)),
            pl.BlockSpec((1, _TB, _H, _W), lambda c, q: (c, q, 0, 0)),
        ],
        out_shape=[
            jax.ShapeDtypeStruct((_C, _NSLOW, _H, _W), jnp.float32),
            jax.ShapeDtypeStruct((_C, _T, _H, _W), jnp.float32),
        ],
    )(frames)
    return (slow, fast)
